# baseline (speedup 1.0000x reference)
import jax
import jax.numpy as jnp
from jax import lax
from jax.experimental import pallas as pl
from jax.experimental.pallas import tpu as pltpu
from jax.experimental.pallas import tpu_sc as plsc

N = 10000
D = 3
K = 32
N_OUT = 32
NW = 32
RPW = 313
NP = NW * RPW
NB = 20
NG = NP // 16
PRIMEG = 64
SEGG = 64
CAPL = 768
STRIDE = CAPL + 16
CAP2 = 512
BIG = 3.0e38


def _body(x0h, x1h, x2h, sqh, cofh, cols_h, vals_h,
          x0v, x1v, x2v, sqv, cofv, bufd, bufc, c2d2, c2col,
          d2row, colrow, colout, valbuf, cntbuf, tbuf):
    wid = lax.axis_index("s") * 2 + lax.axis_index("c")
    r0 = wid * RPW
    row_hi = jnp.minimum(jnp.int32(N), r0 + RPW)

    pltpu.sync_copy(x0h, x0v)
    pltpu.sync_copy(x1h, x1v)
    pltpu.sync_copy(x2h, x2v)
    pltpu.sync_copy(sqh, sqv)
    pltpu.sync_copy(cofh, cofv)

    lane = lax.iota(jnp.int32, 16)
    bigv = jnp.full((16,), BIG, jnp.float32)
    bigiv = jnp.full((16,), 1 << 30, jnp.int32)
    zero16 = jnp.zeros((16,), jnp.int32)
    lanebase = lane * STRIDE
    cof0 = cofv[pl.ds(0, 16)]
    cof1 = cofv[pl.ds(16, 16)]

    def batch_body(b, _):
        rowv = r0 + b * 16 + lane
        valid = rowv < row_hi
        rsafe = jnp.minimum(rowv, NP - 1)
        xi0 = plsc.load_gather(x0v, [rsafe])
        xi1 = plsc.load_gather(x1v, [rsafe])
        xi2 = plsc.load_gather(x2v, [rsafe])
        sqi = plsc.load_gather(sqv, [rsafe])

        def col_d2(j):
            jv = lane * 0 + j
            a0 = plsc.load_gather(x0v, [jv])
            a1 = plsc.load_gather(x1v, [jv])
            a2 = plsc.load_gather(x2v, [jv])
            sj = plsc.load_gather(sqv, [jv])
            mm = (xi0 * a0 + xi1 * a1) + xi2 * a2
            d2 = (sqi + sj) - 2.0 * mm
            return jnp.maximum(d2, 0.0)

        def upd_cls(m1s, m2s, u, d2c):
            hi = jnp.maximum(m1s[u], d2c)
            m1s[u] = jnp.minimum(m1s[u], d2c)
            m2s[u] = jnp.minimum(m2s[u], hi)

        def prime_body(g, carry):
            m1s = list(carry[0])
            m2s = list(carry[1])
            for u in range(16):
                d2c = col_d2(g * 16 + u)
                upd_cls(m1s, m2s, u, d2c)
            return tuple(m1s), tuple(m2s)

        m1s, m2s = lax.fori_loop(
            0, PRIMEG, prime_body,
            (tuple(bigv for _ in range(16)), tuple(bigv for _ in range(16))))

        def tmax(m2s):
            t = m2s[0]
            for u in range(1, 16):
                t = jnp.maximum(t, m2s[u])
            return jnp.where(valid, t, -1.0)

        tv = tmax(m2s)

        def append(d2c, j, cntv, tv):
            mask = d2c <= tv
            colv = lane * 0 + j
            pos = lanebase + cntv
            plsc.store_scatter(bufd, [pos], d2c, mask=mask)
            plsc.store_scatter(bufc, [pos], colv, mask=mask)
            return cntv + mask.astype(jnp.int32)

        def make_seg0(tv):
            def seg0_body(g, cntv):
                for u in range(16):
                    d2c = col_d2(g * 16 + u)
                    cntv = append(d2c, g * 16 + u, cntv, tv)
                return jnp.minimum(cntv, CAPL)
            return seg0_body

        cntv = lax.fori_loop(0, PRIMEG, make_seg0(tv), zero16)

        def make_seg(tv):
            def seg_body(g, carry):
                m1s, m2s, cntv = list(carry[0]), list(carry[1]), carry[2]
                for u in range(16):
                    j = g * 16 + u
                    d2c = col_d2(j)
                    upd_cls(m1s, m2s, u, d2c)
                    cntv = append(d2c, j, cntv, tv)
                return tuple(m1s), tuple(m2s), jnp.minimum(cntv, CAPL)
            return seg_body

        for s in range(1, 10):
            lo = s * SEGG
            hi = min((s + 1) * SEGG, NG)
            m1s, m2s, cntv = lax.fori_loop(lo, hi, make_seg(tv),
                                           (m1s, m2s, cntv))
            tv = tmax(m2s)

        cntbuf[pl.ds(0, 16)] = cntv
        tbuf[pl.ds(0, 16)] = tv

        def row_body(r, _):
            ri = b * 16 + r
            gi = r0 + ri

            @pl.when(gi < row_hi)
            def _row():
                rv = lane * 0 + r
                cnt = plsc.load_gather(cntbuf, [rv])[0]
                tfr = plsc.load_gather(tbuf, [rv])
                base = r * STRIDE

                for v in range((CAP2 + 64) // 16):
                    c2d2[pl.ds(v * 16, 16)] = bigv

                nv4 = (cnt + 63) // 64

                def filt(v4, cnt2):
                    for u in range(4):
                        v = v4 * 4 + u
                        vec = bufd[pl.ds(base + v * 16, 16)]
                        colvec = bufc[pl.ds(base + v * 16, 16)]
                        posv = v * 16 + lane
                        mask = (posv < cnt) & (vec <= tfr)
                        plsc.store_compressed(
                            c2d2.at[pl.ds(cnt2, 16)], vec, mask=mask)
                        plsc.store_compressed(
                            c2col.at[pl.ds(cnt2, 16)], colvec, mask=mask)
                        pc = plsc.all_reduce_population_count(mask)
                        cnt2 = jnp.minimum(cnt2 + pc[0], CAP2)
                    return cnt2

                cnt2 = lax.fori_loop(0, nv4, filt, jnp.int32(0))
                nv24 = (cnt2 + 63) // 64

                def ext(k, _):
                    def mn(v4, carry):
                        m, pm = carry
                        for u in range(4):
                            v = v4 * 4 + u
                            vec = c2d2[pl.ds(v * 16, 16)]
                            posv = v * 16 + lane
                            ltm = vec < m
                            m = jnp.where(ltm, vec, m)
                            pm = jnp.where(ltm, posv, pm)
                        return m, pm

                    m, pm = lax.fori_loop(0, nv24, mn, (bigv, bigiv))
                    mval = jnp.min(m)
                    pmsel = jnp.where(m == mval, pm, bigiv)
                    p = jnp.min(pmsel)
                    pv = lane * 0 + p
                    kv = lane * 0 + k
                    lane0 = lane == 0
                    colv = plsc.load_gather(c2col, [pv])
                    plsc.store_scatter(colrow, [kv], colv, mask=lane0)
                    plsc.store_scatter(d2row, [kv], lane * 0.0 + mval,
                                       mask=lane0)
                    plsc.store_scatter(c2d2, [pv], bigv, mask=lane0)
                    return 0

                lax.fori_loop(0, K, ext, 0)

                colout[pl.ds(ri * K, 16)] = colrow[pl.ds(0, 16)]
                colout[pl.ds(ri * K + 16, 16)] = colrow[pl.ds(16, 16)]

                rb = lax.rem(ri, 8)

                def vk(k, _):
                    d2k = plsc.load_gather(d2row, [lane * 0 + k])
                    valbuf[rb * K + k, pl.ds(0, 16)] = jnp.exp(d2k * cof0)
                    valbuf[rb * K + k, pl.ds(16, 16)] = jnp.exp(d2k * cof1)
                    return 0

                lax.fori_loop(0, K, vk, 0, unroll=4)

            @pl.when((lax.rem(ri, 8) == 7) & (ri < RPW))
            def _flush():
                vbase = (r0 + ri - 7) * K
                pltpu.sync_copy(valbuf, vals_h.at[pl.ds(vbase, 8 * K)])

            return 0

        lax.fori_loop(0, 16, row_body, 0)
        return 0

    lax.fori_loop(0, NB, batch_body, 0)
    pltpu.sync_copy(valbuf.at[pl.ds(0, K)],
                    vals_h.at[pl.ds((r0 + RPW - 1) * K, K)])
    pltpu.sync_copy(colout, cols_h.at[pl.ds(r0 * K, RPW * K)])


@jax.jit
def _run(x0, x1, x2, sqp, cof):
    mesh = plsc.VectorSubcoreMesh(core_axis_name="c", subcore_axis_name="s")
    f = pl.kernel(
        _body,
        out_type=(
            jax.ShapeDtypeStruct((NP * K,), jnp.int32),
            jax.ShapeDtypeStruct((NP * K, N_OUT), jnp.float32),
        ),
        mesh=mesh,
        compiler_params=pltpu.CompilerParams(needs_layout_passes=False),
        scratch_types=[
            pltpu.VMEM((NP,), jnp.float32),
            pltpu.VMEM((NP,), jnp.float32),
            pltpu.VMEM((NP,), jnp.float32),
            pltpu.VMEM((NP,), jnp.float32),
            pltpu.VMEM((N_OUT,), jnp.float32),
            pltpu.VMEM((16 * STRIDE,), jnp.float32),
            pltpu.VMEM((16 * STRIDE,), jnp.int32),
            pltpu.VMEM((CAP2 + 64,), jnp.float32),
            pltpu.VMEM((CAP2 + 64,), jnp.int32),
            pltpu.VMEM((K,), jnp.float32),
            pltpu.VMEM((K,), jnp.int32),
            pltpu.VMEM((RPW * K,), jnp.int32),
            pltpu.VMEM((8 * K, N_OUT), jnp.float32),
            pltpu.VMEM((16,), jnp.int32),
            pltpu.VMEM((16,), jnp.float32),
        ],
    )
    return f(x0, x1, x2, sqp, cof)


def kernel(input_coord):
    x = input_coord
    sq = jnp.sum(x * x, axis=-1)
    u = lax.bitcast_convert_type(x, jnp.uint32)
    r = u + jnp.uint32(0x7FFF) + ((u >> 16) & jnp.uint32(1))
    xb = lax.bitcast_convert_type(r & jnp.uint32(0xFFFF0000), jnp.float32)

    padc = jnp.zeros((NP - N,), jnp.float32)
    x0 = jnp.concatenate([xb[:, 0], padc])
    x1 = jnp.concatenate([xb[:, 1], padc])
    x2 = jnp.concatenate([xb[:, 2], padc])
    sqp = jnp.concatenate([sq, jnp.full((NP - N,), BIG, jnp.float32)])

    sig = jnp.linspace(0.5, 5.0, N_OUT).astype(jnp.float32)
    cof = -1.0 / (2.0 * sig * sig)

    cols, vals = _run(x0, x1, x2, sqp, cof)

    row = jnp.repeat(jnp.arange(N, dtype=jnp.int64), K)
    col = cols[: N * K].astype(jnp.int64)
    indices = jnp.stack([row, col], axis=0)
    values = vals[: N * K]
    return indices, values


# speedup vs baseline: 1.7559x; 1.7559x over previous
import jax
import jax.numpy as jnp
from jax import lax
from jax.experimental import pallas as pl
from jax.experimental.pallas import tpu as pltpu
from jax.experimental.pallas import tpu_sc as plsc

N = 10000
D = 3
K = 32
N_OUT = 32
NW = 32
RPW = 313
NP = NW * RPW
NB = 20
NG = NP // 16
PRIMEG = 64
SEGG = 64
CAPL = 768
STRIDE = CAPL + 16
CAP2 = 512
BIG = 3.0e38


def _body(x0h, x1h, x2h, sqh, cofh, cols_h, vals_h,
          x0v, x1v, x2v, sqv, cofv, bufd, bufc, c2d2, c2col,
          d2row, colrow, colout, valbuf, cntbuf, tbuf, m1arr, m2arr):
    wid = lax.axis_index("s") * 2 + lax.axis_index("c")
    r0 = wid * RPW
    row_hi = jnp.minimum(jnp.int32(N), r0 + RPW)

    pltpu.sync_copy(x0h, x0v)
    pltpu.sync_copy(x1h, x1v)
    pltpu.sync_copy(x2h, x2v)
    pltpu.sync_copy(sqh, sqv)
    pltpu.sync_copy(cofh, cofv)

    lane = lax.iota(jnp.int32, 16)
    bigv = jnp.full((16,), BIG, jnp.float32)
    bigiv = jnp.full((16,), 1 << 30, jnp.int32)
    zero16 = jnp.zeros((16,), jnp.int32)
    lanebase = lane * STRIDE
    cof0 = cofv[pl.ds(0, 16)]
    cof1 = cofv[pl.ds(16, 16)]

    ubc = [jnp.full((16,), u, jnp.int32) for u in range(16)]

    def batch_body(b, _):
        rowv = r0 + b * 16 + lane
        valid = rowv < row_hi
        rsafe = jnp.minimum(rowv, NP - 1)
        xi0 = plsc.load_gather(x0v, [rsafe])
        xi1 = plsc.load_gather(x1v, [rsafe])
        xi2 = plsc.load_gather(x2v, [rsafe])
        sqi = plsc.load_gather(sqv, [rsafe])

        for u in range(16):
            m1arr[pl.ds(u * 16, 16)] = bigv
            m2arr[pl.ds(u * 16, 16)] = bigv

        def group_d2(g):
            gb = g * 16
            a0g = x0v[pl.ds(gb, 16)]
            a1g = x1v[pl.ds(gb, 16)]
            a2g = x2v[pl.ds(gb, 16)]
            sjg = sqv[pl.ds(gb, 16)]

            def one(u):
                a0 = jnp.take_along_axis(a0g, ubc[u], axis=0)
                a1 = jnp.take_along_axis(a1g, ubc[u], axis=0)
                a2 = jnp.take_along_axis(a2g, ubc[u], axis=0)
                sj = jnp.take_along_axis(sjg, ubc[u], axis=0)
                mm = (xi0 * a0 + xi1 * a1) + xi2 * a2
                d2 = (sqi + sj) - 2.0 * mm
                return jnp.maximum(d2, 0.0)

            return one

        def upd_cls(u, d2c):
            m1 = m1arr[pl.ds(u * 16, 16)]
            m2 = m2arr[pl.ds(u * 16, 16)]
            hi = jnp.maximum(m1, d2c)
            m1arr[pl.ds(u * 16, 16)] = jnp.minimum(m1, d2c)
            m2arr[pl.ds(u * 16, 16)] = jnp.minimum(m2, hi)

        def prime_body(g, _):
            one = group_d2(g)
            for u in range(16):
                upd_cls(u, one(u))
            return 0

        lax.fori_loop(0, PRIMEG, prime_body, 0)

        def tmax():
            t = m2arr[pl.ds(0, 16)]
            for u in range(1, 16):
                t = jnp.maximum(t, m2arr[pl.ds(u * 16, 16)])
            return jnp.where(valid, t, -1.0)

        tv = tmax()

        def append(d2c, j, cntv, tv):
            mask = d2c <= tv
            colv = lane * 0 + j
            pos = lanebase + cntv
            plsc.store_scatter(bufd, [pos], d2c, mask=mask)
            plsc.store_scatter(bufc, [pos], colv, mask=mask)
            return cntv + mask.astype(jnp.int32)

        def make_seg0(tv):
            def seg0_body(g, cntv):
                one = group_d2(g)
                for u in range(16):
                    cntv = append(one(u), g * 16 + u, cntv, tv)
                return jnp.minimum(cntv, CAPL)
            return seg0_body

        cntv = lax.fori_loop(0, PRIMEG, make_seg0(tv), zero16)

        def make_seg(tv):
            def seg_body(g, cntv):
                one = group_d2(g)
                for u in range(16):
                    d2c = one(u)
                    upd_cls(u, d2c)
                    cntv = append(d2c, g * 16 + u, cntv, tv)
                return jnp.minimum(cntv, CAPL)
            return seg_body

        for s in range(1, 10):
            lo = s * SEGG
            hi = min((s + 1) * SEGG, NG)
            cntv = lax.fori_loop(lo, hi, make_seg(tv), cntv)
            tv = tmax()

        cntbuf[pl.ds(0, 16)] = cntv
        tbuf[pl.ds(0, 16)] = tv

        def row_body(r, _):
            ri = b * 16 + r
            gi = r0 + ri

            @pl.when(gi < row_hi)
            def _row():
                rv = lane * 0 + r
                cnt = plsc.load_gather(cntbuf, [rv])[0]
                tfr = plsc.load_gather(tbuf, [rv])
                base = r * STRIDE

                for v in range((CAP2 + 64) // 16):
                    c2d2[pl.ds(v * 16, 16)] = bigv

                nv4 = (cnt + 63) // 64

                def filt(v4, cnt2):
                    for u in range(4):
                        v = v4 * 4 + u
                        vec = bufd[pl.ds(base + v * 16, 16)]
                        colvec = bufc[pl.ds(base + v * 16, 16)]
                        posv = v * 16 + lane
                        mask = (posv < cnt) & (vec <= tfr)
                        plsc.store_compressed(
                            c2d2.at[pl.ds(cnt2, 16)], vec, mask=mask)
                        plsc.store_compressed(
                            c2col.at[pl.ds(cnt2, 16)], colvec, mask=mask)
                        pc = plsc.all_reduce_population_count(mask)
                        cnt2 = jnp.minimum(cnt2 + pc[0], CAP2)
                    return cnt2

                cnt2 = lax.fori_loop(0, nv4, filt, jnp.int32(0))
                nv24 = (cnt2 + 63) // 64

                def ext(k, _):
                    def mn(v4, carry):
                        m, pm = carry
                        for u in range(4):
                            v = v4 * 4 + u
                            vec = c2d2[pl.ds(v * 16, 16)]
                            posv = v * 16 + lane
                            ltm = vec < m
                            m = jnp.where(ltm, vec, m)
                            pm = jnp.where(ltm, posv, pm)
                        return m, pm

                    m, pm = lax.fori_loop(0, nv24, mn, (bigv, bigiv))
                    mval = jnp.min(m)
                    pmsel = jnp.where(m == mval, pm, bigiv)
                    p = jnp.min(pmsel)
                    pv = lane * 0 + p
                    kv = lane * 0 + k
                    lane0 = lane == 0
                    colv = plsc.load_gather(c2col, [pv])
                    plsc.store_scatter(colrow, [kv], colv, mask=lane0)
                    plsc.store_scatter(d2row, [kv], lane * 0.0 + mval,
                                       mask=lane0)
                    plsc.store_scatter(c2d2, [pv], bigv, mask=lane0)
                    return 0

                lax.fori_loop(0, K, ext, 0)

                colout[pl.ds(ri * K, 16)] = colrow[pl.ds(0, 16)]
                colout[pl.ds(ri * K + 16, 16)] = colrow[pl.ds(16, 16)]

                rb = lax.rem(ri, 8)

                def vk(k, _):
                    d2k = plsc.load_gather(d2row, [lane * 0 + k])
                    valbuf[rb * K + k, pl.ds(0, 16)] = jnp.exp(d2k * cof0)
                    valbuf[rb * K + k, pl.ds(16, 16)] = jnp.exp(d2k * cof1)
                    return 0

                lax.fori_loop(0, K, vk, 0, unroll=4)

            @pl.when((lax.rem(ri, 8) == 7) & (ri < RPW))
            def _flush():
                vbase = (r0 + ri - 7) * K
                pltpu.sync_copy(valbuf, vals_h.at[pl.ds(vbase, 8 * K)])

            return 0

        lax.fori_loop(0, 16, row_body, 0)
        return 0

    lax.fori_loop(0, NB, batch_body, 0)
    pltpu.sync_copy(valbuf.at[pl.ds(0, K)],
                    vals_h.at[pl.ds((r0 + RPW - 1) * K, K)])
    pltpu.sync_copy(colout, cols_h.at[pl.ds(r0 * K, RPW * K)])


@jax.jit
def _run(x0, x1, x2, sqp, cof):
    mesh = plsc.VectorSubcoreMesh(core_axis_name="c", subcore_axis_name="s")
    f = pl.kernel(
        _body,
        out_type=(
            jax.ShapeDtypeStruct((NP * K,), jnp.int32),
            jax.ShapeDtypeStruct((NP * K, N_OUT), jnp.float32),
        ),
        mesh=mesh,
        compiler_params=pltpu.CompilerParams(needs_layout_passes=False),
        scratch_types=[
            pltpu.VMEM((NP,), jnp.float32),
            pltpu.VMEM((NP,), jnp.float32),
            pltpu.VMEM((NP,), jnp.float32),
            pltpu.VMEM((NP,), jnp.float32),
            pltpu.VMEM((N_OUT,), jnp.float32),
            pltpu.VMEM((16 * STRIDE,), jnp.float32),
            pltpu.VMEM((16 * STRIDE,), jnp.int32),
            pltpu.VMEM((CAP2 + 64,), jnp.float32),
            pltpu.VMEM((CAP2 + 64,), jnp.int32),
            pltpu.VMEM((K,), jnp.float32),
            pltpu.VMEM((K,), jnp.int32),
            pltpu.VMEM((RPW * K,), jnp.int32),
            pltpu.VMEM((8 * K, N_OUT), jnp.float32),
            pltpu.VMEM((16,), jnp.int32),
            pltpu.VMEM((16,), jnp.float32),
            pltpu.VMEM((256,), jnp.float32),
            pltpu.VMEM((256,), jnp.float32),
        ],
    )
    return f(x0, x1, x2, sqp, cof)


def kernel(input_coord):
    x = input_coord
    sq = jnp.sum(x * x, axis=-1)
    u = lax.bitcast_convert_type(x, jnp.uint32)
    r = u + jnp.uint32(0x7FFF) + ((u >> 16) & jnp.uint32(1))
    xb = lax.bitcast_convert_type(r & jnp.uint32(0xFFFF0000), jnp.float32)

    padc = jnp.zeros((NP - N,), jnp.float32)
    x0 = jnp.concatenate([xb[:, 0], padc])
    x1 = jnp.concatenate([xb[:, 1], padc])
    x2 = jnp.concatenate([xb[:, 2], padc])
    sqp = jnp.concatenate([sq, jnp.full((NP - N,), BIG, jnp.float32)])

    sig = jnp.linspace(0.5, 5.0, N_OUT).astype(jnp.float32)
    cof = -1.0 / (2.0 * sig * sig)

    cols, vals = _run(x0, x1, x2, sqp, cof)

    row = jnp.repeat(jnp.arange(N, dtype=jnp.int64), K)
    col = cols[: N * K].astype(jnp.int64)
    indices = jnp.stack([row, col], axis=0)
    values = vals[: N * K]
    return indices, values


# speedup vs baseline: 1.8404x; 1.0481x over previous
import jax
import jax.numpy as jnp
from jax import lax
from jax.experimental import pallas as pl
from jax.experimental.pallas import tpu as pltpu
from jax.experimental.pallas import tpu_sc as plsc

N = 10000
D = 3
K = 32
N_OUT = 32
NW = 32
RPW = 313
NP = NW * RPW
NB = 20
NG = NP // 16
PRIMEG = 64
SEGG = 64
CAPL = 768
STRIDE = CAPL + 16
CAP2 = 512
BIG = 3.0e38


def _body(x0h, x1h, x2h, sqh, cofh, cols_h, vals_h,
          x0v, x1v, x2v, sqv, cofv, bufd, bufc, c2d2, c2col,
          d2row, colrow, colout, valbuf, cntbuf, tbuf, m1arr, m2arr):
    wid = lax.axis_index("s") * 2 + lax.axis_index("c")
    r0 = wid * RPW
    row_hi = jnp.minimum(jnp.int32(N), r0 + RPW)

    pltpu.sync_copy(x0h, x0v)
    pltpu.sync_copy(x1h, x1v)
    pltpu.sync_copy(x2h, x2v)
    pltpu.sync_copy(sqh, sqv)
    pltpu.sync_copy(cofh, cofv)

    lane = lax.iota(jnp.int32, 16)
    bigv = jnp.full((16,), BIG, jnp.float32)
    bigiv = jnp.full((16,), 1 << 30, jnp.int32)
    zero16 = jnp.zeros((16,), jnp.int32)
    lanebase = lane * STRIDE
    cof0 = cofv[pl.ds(0, 16)]
    cof1 = cofv[pl.ds(16, 16)]

    ubc = [jnp.full((16,), u, jnp.int32) for u in range(16)]

    def batch_body(b, _):
        rowv = r0 + b * 16 + lane
        valid = rowv < row_hi
        rsafe = jnp.minimum(rowv, NP - 1)
        xi0 = 2.0 * plsc.load_gather(x0v, [rsafe])
        xi1 = 2.0 * plsc.load_gather(x1v, [rsafe])
        xi2 = 2.0 * plsc.load_gather(x2v, [rsafe])
        sqi = plsc.load_gather(sqv, [rsafe])

        for u in range(16):
            m1arr[pl.ds(u * 16, 16)] = bigv
            m2arr[pl.ds(u * 16, 16)] = bigv

        def group_d2(g):
            gb = g * 16
            a0g = x0v[pl.ds(gb, 16)]
            a1g = x1v[pl.ds(gb, 16)]
            a2g = x2v[pl.ds(gb, 16)]
            sjg = sqv[pl.ds(gb, 16)]

            def one(u):
                a0 = jnp.take_along_axis(a0g, ubc[u], axis=0)
                a1 = jnp.take_along_axis(a1g, ubc[u], axis=0)
                a2 = jnp.take_along_axis(a2g, ubc[u], axis=0)
                sj = jnp.take_along_axis(sjg, ubc[u], axis=0)
                mm2 = (xi0 * a0 + xi1 * a1) + xi2 * a2
                d2 = (sqi + sj) - mm2
                return jnp.maximum(d2, 0.0)

            return one

        def upd_cls(u, d2c):
            m1 = m1arr[pl.ds(u * 16, 16)]
            m2 = m2arr[pl.ds(u * 16, 16)]
            hi = jnp.maximum(m1, d2c)
            m1arr[pl.ds(u * 16, 16)] = jnp.minimum(m1, d2c)
            m2arr[pl.ds(u * 16, 16)] = jnp.minimum(m2, hi)

        def prime_body(g, _):
            one = group_d2(g)
            for u in range(16):
                upd_cls(u, one(u))
            return 0

        lax.fori_loop(0, PRIMEG, prime_body, 0)

        def tmax():
            t = m2arr[pl.ds(0, 16)]
            for u in range(1, 16):
                t = jnp.maximum(t, m2arr[pl.ds(u * 16, 16)])
            return jnp.where(valid, t, -1.0)

        tv = tmax()

        def append(d2c, j, posv, tv):
            mask = d2c <= tv
            colv = lane * 0 + j
            plsc.store_scatter(bufd, [posv], d2c, mask=mask)
            plsc.store_scatter(bufc, [posv], colv, mask=mask)
            return posv + mask.astype(jnp.int32)

        capp = lanebase + CAPL

        def make_seg0(tv):
            def seg0_body(g, posv):
                one = group_d2(g)
                for u in range(16):
                    posv = append(one(u), g * 16 + u, posv, tv)
                return jnp.minimum(posv, capp)
            return seg0_body

        posv = lax.fori_loop(0, PRIMEG, make_seg0(tv), lanebase)

        def make_seg(tv):
            def seg_body(g, posv):
                one = group_d2(g)
                for u in range(16):
                    d2c = one(u)
                    upd_cls(u, d2c)
                    posv = append(d2c, g * 16 + u, posv, tv)
                return jnp.minimum(posv, capp)
            return seg_body

        for s in range(1, 10):
            lo = s * SEGG
            hi = min((s + 1) * SEGG, NG)
            posv = lax.fori_loop(lo, hi, make_seg(tv), posv)
            tv = tmax()

        cntbuf[pl.ds(0, 16)] = posv - lanebase
        tbuf[pl.ds(0, 16)] = tv

        def row_body(r, _):
            ri = b * 16 + r
            gi = r0 + ri

            @pl.when(gi < row_hi)
            def _row():
                rv = lane * 0 + r
                cnt = plsc.load_gather(cntbuf, [rv])[0]
                tfr = plsc.load_gather(tbuf, [rv])
                base = r * STRIDE

                for v in range((CAP2 + 64) // 16):
                    c2d2[pl.ds(v * 16, 16)] = bigv

                nv4 = (cnt + 63) // 64

                def filt(v4, cnt2):
                    for u in range(4):
                        v = v4 * 4 + u
                        vec = bufd[pl.ds(base + v * 16, 16)]
                        colvec = bufc[pl.ds(base + v * 16, 16)]
                        posv = v * 16 + lane
                        mask = (posv < cnt) & (vec <= tfr)
                        plsc.store_compressed(
                            c2d2.at[pl.ds(cnt2, 16)], vec, mask=mask)
                        plsc.store_compressed(
                            c2col.at[pl.ds(cnt2, 16)], colvec, mask=mask)
                        pc = plsc.all_reduce_population_count(mask)
                        cnt2 = jnp.minimum(cnt2 + pc[0], CAP2)
                    return cnt2

                cnt2 = lax.fori_loop(0, nv4, filt, jnp.int32(0))
                nv24 = (cnt2 + 63) // 64

                def ext(k, _):
                    def mn(v4, carry):
                        m, pm = carry
                        for u in range(4):
                            v = v4 * 4 + u
                            vec = c2d2[pl.ds(v * 16, 16)]
                            posv = v * 16 + lane
                            ltm = vec < m
                            m = jnp.where(ltm, vec, m)
                            pm = jnp.where(ltm, posv, pm)
                        return m, pm

                    m, pm = lax.fori_loop(0, nv24, mn, (bigv, bigiv))
                    for d in (1, 2, 4, 8):
                        ms = jnp.take_along_axis(m, lane ^ d, axis=0)
                        pms = jnp.take_along_axis(pm, lane ^ d, axis=0)
                        ltm = (ms < m) | ((ms == m) & (pms < pm))
                        m = jnp.where(ltm, ms, m)
                        pm = jnp.where(ltm, pms, pm)
                    kv = lane * 0 + k
                    lane0 = lane == 0
                    colv = plsc.load_gather(c2col, [pm])
                    plsc.store_scatter(colrow, [kv], colv, mask=lane0)
                    plsc.store_scatter(d2row, [kv], m, mask=lane0)
                    plsc.store_scatter(c2d2, [pm], bigv, mask=lane0)
                    return 0

                lax.fori_loop(0, K, ext, 0)

                colout[pl.ds(ri * K, 16)] = colrow[pl.ds(0, 16)]
                colout[pl.ds(ri * K + 16, 16)] = colrow[pl.ds(16, 16)]

                rb = lax.rem(ri, 8)

                def vk(k, _):
                    d2k = plsc.load_gather(d2row, [lane * 0 + k])
                    valbuf[rb * K + k, pl.ds(0, 16)] = jnp.exp(d2k * cof0)
                    valbuf[rb * K + k, pl.ds(16, 16)] = jnp.exp(d2k * cof1)
                    return 0

                lax.fori_loop(0, K, vk, 0, unroll=4)

            @pl.when((lax.rem(ri, 8) == 7) & (ri < RPW))
            def _flush():
                vbase = (r0 + ri - 7) * K
                pltpu.sync_copy(valbuf, vals_h.at[pl.ds(vbase, 8 * K)])

            return 0

        lax.fori_loop(0, 16, row_body, 0)
        return 0

    lax.fori_loop(0, NB, batch_body, 0)
    pltpu.sync_copy(valbuf.at[pl.ds(0, K)],
                    vals_h.at[pl.ds((r0 + RPW - 1) * K, K)])
    pltpu.sync_copy(colout, cols_h.at[pl.ds(r0 * K, RPW * K)])


@jax.jit
def _run(x0, x1, x2, sqp, cof):
    mesh = plsc.VectorSubcoreMesh(core_axis_name="c", subcore_axis_name="s")
    f = pl.kernel(
        _body,
        out_type=(
            jax.ShapeDtypeStruct((NP * K,), jnp.int32),
            jax.ShapeDtypeStruct((NP * K, N_OUT), jnp.float32),
        ),
        mesh=mesh,
        compiler_params=pltpu.CompilerParams(needs_layout_passes=False),
        scratch_types=[
            pltpu.VMEM((NP,), jnp.float32),
            pltpu.VMEM((NP,), jnp.float32),
            pltpu.VMEM((NP,), jnp.float32),
            pltpu.VMEM((NP,), jnp.float32),
            pltpu.VMEM((N_OUT,), jnp.float32),
            pltpu.VMEM((16 * STRIDE,), jnp.float32),
            pltpu.VMEM((16 * STRIDE,), jnp.int32),
            pltpu.VMEM((CAP2 + 64,), jnp.float32),
            pltpu.VMEM((CAP2 + 64,), jnp.int32),
            pltpu.VMEM((K,), jnp.float32),
            pltpu.VMEM((K,), jnp.int32),
            pltpu.VMEM((RPW * K,), jnp.int32),
            pltpu.VMEM((8 * K, N_OUT), jnp.float32),
            pltpu.VMEM((16,), jnp.int32),
            pltpu.VMEM((16,), jnp.float32),
            pltpu.VMEM((256,), jnp.float32),
            pltpu.VMEM((256,), jnp.float32),
        ],
    )
    return f(x0, x1, x2, sqp, cof)


def kernel(input_coord):
    x = input_coord
    sq = jnp.sum(x * x, axis=-1)
    u = lax.bitcast_convert_type(x, jnp.uint32)
    r = u + jnp.uint32(0x7FFF) + ((u >> 16) & jnp.uint32(1))
    xb = lax.bitcast_convert_type(r & jnp.uint32(0xFFFF0000), jnp.float32)

    padc = jnp.zeros((NP - N,), jnp.float32)
    x0 = jnp.concatenate([xb[:, 0], padc])
    x1 = jnp.concatenate([xb[:, 1], padc])
    x2 = jnp.concatenate([xb[:, 2], padc])
    sqp = jnp.concatenate([sq, jnp.full((NP - N,), BIG, jnp.float32)])

    sig = jnp.linspace(0.5, 5.0, N_OUT).astype(jnp.float32)
    cof = -1.0 / (2.0 * sig * sig)

    cols, vals = _run(x0, x1, x2, sqp, cof)

    row = jnp.repeat(jnp.arange(N, dtype=jnp.int64), K)
    col = cols[: N * K].astype(jnp.int64)
    indices = jnp.stack([row, col], axis=0)
    values = vals[: N * K]
    return indices, values


# speedup vs baseline: 1.8736x; 1.0181x over previous
import jax
import jax.numpy as jnp
from jax import lax
from jax.experimental import pallas as pl
from jax.experimental.pallas import tpu as pltpu
from jax.experimental.pallas import tpu_sc as plsc

N = 10000
D = 3
K = 32
N_OUT = 32
NW = 32
RPW = 313
NP = NW * RPW
NB = 20
NG = NP // 16
PRIMEG = 64
SEGG = 64
CAPL = 1024
STRIDE = CAPL + 16
CAP2 = 512
BIG = 3.0e38


def _body(x0h, x1h, x2h, sqh, cofh, cols_h, vals_h,
          x0v, x1v, x2v, sqv, cofv, bufd, bufc, c2d2, c2col,
          d2row, colrow, colout, valbuf, cntbuf, tbuf, m1arr):
    wid = lax.axis_index("s") * 2 + lax.axis_index("c")
    r0 = wid * RPW
    row_hi = jnp.minimum(jnp.int32(N), r0 + RPW)

    pltpu.sync_copy(x0h, x0v)
    pltpu.sync_copy(x1h, x1v)
    pltpu.sync_copy(x2h, x2v)
    pltpu.sync_copy(sqh, sqv)
    pltpu.sync_copy(cofh, cofv)

    lane = lax.iota(jnp.int32, 16)
    bigv = jnp.full((16,), BIG, jnp.float32)
    bigiv = jnp.full((16,), 1 << 30, jnp.int32)
    zero16 = jnp.zeros((16,), jnp.int32)
    lanebase = lane * STRIDE
    cof0 = cofv[pl.ds(0, 16)]
    cof1 = cofv[pl.ds(16, 16)]

    ubc = [jnp.full((16,), u, jnp.int32) for u in range(16)]

    def batch_body(b, _):
        rowv = r0 + b * 16 + lane
        valid = rowv < row_hi
        rsafe = jnp.minimum(rowv, NP - 1)
        xi0 = 2.0 * plsc.load_gather(x0v, [rsafe])
        xi1 = 2.0 * plsc.load_gather(x1v, [rsafe])
        xi2 = 2.0 * plsc.load_gather(x2v, [rsafe])
        sqi = plsc.load_gather(sqv, [rsafe])

        for u in range(32):
            m1arr[pl.ds(u * 16, 16)] = bigv

        def group_d2(g):
            gb = g * 16
            a0g = x0v[pl.ds(gb, 16)]
            a1g = x1v[pl.ds(gb, 16)]
            a2g = x2v[pl.ds(gb, 16)]
            sjg = sqv[pl.ds(gb, 16)]

            def one(u):
                a0 = jnp.take_along_axis(a0g, ubc[u], axis=0)
                a1 = jnp.take_along_axis(a1g, ubc[u], axis=0)
                a2 = jnp.take_along_axis(a2g, ubc[u], axis=0)
                sj = jnp.take_along_axis(sjg, ubc[u], axis=0)
                mm2 = (xi0 * a0 + xi1 * a1) + xi2 * a2
                d2 = (sqi + sj) - mm2
                return jnp.maximum(d2, 0.0)

            return one

        def upd_cls(goff, u, d2c):
            m1 = m1arr[pl.ds(goff + u * 16, 16)]
            m1arr[pl.ds(goff + u * 16, 16)] = jnp.minimum(m1, d2c)

        def goff_of(g):
            return (g & 1) * 256

        def prime_body(g, _):
            one = group_d2(g)
            goff = goff_of(g)
            for u in range(16):
                upd_cls(goff, u, one(u))
            return 0

        lax.fori_loop(0, PRIMEG, prime_body, 0)

        def tmax():
            t = m1arr[pl.ds(0, 16)]
            for u in range(1, 32):
                t = jnp.maximum(t, m1arr[pl.ds(u * 16, 16)])
            return jnp.where(valid, t, -1.0)

        tv = tmax()

        def append(d2c, j, posv, tv):
            mask = d2c <= tv
            colv = lane * 0 + j
            plsc.store_scatter(bufd, [posv], d2c, mask=mask)
            plsc.store_scatter(bufc, [posv], colv, mask=mask)
            return posv + mask.astype(jnp.int32)

        capp = lanebase + CAPL

        def make_seg0(tv):
            def seg0_body(g, posv):
                one = group_d2(g)
                for u in range(16):
                    posv = append(one(u), g * 16 + u, posv, tv)
                return jnp.minimum(posv, capp)
            return seg0_body

        posv = lax.fori_loop(0, PRIMEG, make_seg0(tv), lanebase)

        def make_seg(tv):
            def seg_body(g, posv):
                one = group_d2(g)
                goff = goff_of(g)
                for u in range(16):
                    d2c = one(u)
                    upd_cls(goff, u, d2c)
                    posv = append(d2c, g * 16 + u, posv, tv)
                return jnp.minimum(posv, capp)
            return seg_body

        for s in range(1, 10):
            lo = s * SEGG
            hi = min((s + 1) * SEGG, NG)
            posv = lax.fori_loop(lo, hi, make_seg(tv), posv)
            tv = tmax()

        cntbuf[pl.ds(0, 16)] = posv - lanebase
        tbuf[pl.ds(0, 16)] = tv

        def row_body(r, _):
            ri = b * 16 + r
            gi = r0 + ri

            @pl.when(gi < row_hi)
            def _row():
                rv = lane * 0 + r
                cnt = plsc.load_gather(cntbuf, [rv])[0]
                tfr = plsc.load_gather(tbuf, [rv])
                base = r * STRIDE

                for v in range((CAP2 + 64) // 16):
                    c2d2[pl.ds(v * 16, 16)] = bigv

                nv4 = (cnt + 63) // 64

                def filt(v4, cnt2):
                    for u in range(4):
                        v = v4 * 4 + u
                        vec = bufd[pl.ds(base + v * 16, 16)]
                        colvec = bufc[pl.ds(base + v * 16, 16)]
                        posv = v * 16 + lane
                        mask = (posv < cnt) & (vec <= tfr)
                        plsc.store_compressed(
                            c2d2.at[pl.ds(cnt2, 16)], vec, mask=mask)
                        plsc.store_compressed(
                            c2col.at[pl.ds(cnt2, 16)], colvec, mask=mask)
                        pc = plsc.all_reduce_population_count(mask)
                        cnt2 = jnp.minimum(cnt2 + pc[0], CAP2)
                    return cnt2

                cnt2 = lax.fori_loop(0, nv4, filt, jnp.int32(0))
                nv24 = (cnt2 + 63) // 64

                def ext(k, _):
                    def mn(v4, carry):
                        m, pm = carry
                        for u in range(4):
                            v = v4 * 4 + u
                            vec = c2d2[pl.ds(v * 16, 16)]
                            posv = v * 16 + lane
                            ltm = vec < m
                            m = jnp.where(ltm, vec, m)
                            pm = jnp.where(ltm, posv, pm)
                        return m, pm

                    m, pm = lax.fori_loop(0, nv24, mn, (bigv, bigiv))
                    for d in (1, 2, 4, 8):
                        ms = jnp.take_along_axis(m, lane ^ d, axis=0)
                        pms = jnp.take_along_axis(pm, lane ^ d, axis=0)
                        ltm = (ms < m) | ((ms == m) & (pms < pm))
                        m = jnp.where(ltm, ms, m)
                        pm = jnp.where(ltm, pms, pm)
                    kv = lane * 0 + k
                    lane0 = lane == 0
                    colv = plsc.load_gather(c2col, [pm])
                    plsc.store_scatter(colrow, [kv], colv, mask=lane0)
                    plsc.store_scatter(d2row, [kv], m, mask=lane0)
                    plsc.store_scatter(c2d2, [pm], bigv, mask=lane0)
                    return 0

                lax.fori_loop(0, K, ext, 0)

                colout[pl.ds(ri * K, 16)] = colrow[pl.ds(0, 16)]
                colout[pl.ds(ri * K + 16, 16)] = colrow[pl.ds(16, 16)]

                rb = lax.rem(ri, 8)

                def vk(k, _):
                    d2k = plsc.load_gather(d2row, [lane * 0 + k])
                    valbuf[rb * K + k, pl.ds(0, 16)] = jnp.exp(d2k * cof0)
                    valbuf[rb * K + k, pl.ds(16, 16)] = jnp.exp(d2k * cof1)
                    return 0

                lax.fori_loop(0, K, vk, 0, unroll=4)

            @pl.when((lax.rem(ri, 8) == 7) & (ri < RPW))
            def _flush():
                vbase = (r0 + ri - 7) * K
                pltpu.sync_copy(valbuf, vals_h.at[pl.ds(vbase, 8 * K)])

            return 0

        lax.fori_loop(0, 16, row_body, 0)
        return 0

    lax.fori_loop(0, NB, batch_body, 0)
    pltpu.sync_copy(valbuf.at[pl.ds(0, K)],
                    vals_h.at[pl.ds((r0 + RPW - 1) * K, K)])
    pltpu.sync_copy(colout, cols_h.at[pl.ds(r0 * K, RPW * K)])


@jax.jit
def _run(x0, x1, x2, sqp, cof):
    mesh = plsc.VectorSubcoreMesh(core_axis_name="c", subcore_axis_name="s")
    f = pl.kernel(
        _body,
        out_type=(
            jax.ShapeDtypeStruct((NP * K,), jnp.int32),
            jax.ShapeDtypeStruct((NP * K, N_OUT), jnp.float32),
        ),
        mesh=mesh,
        compiler_params=pltpu.CompilerParams(needs_layout_passes=False),
        scratch_types=[
            pltpu.VMEM((NP,), jnp.float32),
            pltpu.VMEM((NP,), jnp.float32),
            pltpu.VMEM((NP,), jnp.float32),
            pltpu.VMEM((NP,), jnp.float32),
            pltpu.VMEM((N_OUT,), jnp.float32),
            pltpu.VMEM((16 * STRIDE,), jnp.float32),
            pltpu.VMEM((16 * STRIDE,), jnp.int32),
            pltpu.VMEM((CAP2 + 64,), jnp.float32),
            pltpu.VMEM((CAP2 + 64,), jnp.int32),
            pltpu.VMEM((K,), jnp.float32),
            pltpu.VMEM((K,), jnp.int32),
            pltpu.VMEM((RPW * K,), jnp.int32),
            pltpu.VMEM((8 * K, N_OUT), jnp.float32),
            pltpu.VMEM((16,), jnp.int32),
            pltpu.VMEM((16,), jnp.float32),
            pltpu.VMEM((512,), jnp.float32),
        ],
    )
    return f(x0, x1, x2, sqp, cof)


def kernel(input_coord):
    x = input_coord
    sq = jnp.sum(x * x, axis=-1)
    u = lax.bitcast_convert_type(x, jnp.uint32)
    r = u + jnp.uint32(0x7FFF) + ((u >> 16) & jnp.uint32(1))
    xb = lax.bitcast_convert_type(r & jnp.uint32(0xFFFF0000), jnp.float32)

    padc = jnp.zeros((NP - N,), jnp.float32)
    x0 = jnp.concatenate([xb[:, 0], padc])
    x1 = jnp.concatenate([xb[:, 1], padc])
    x2 = jnp.concatenate([xb[:, 2], padc])
    sqp = jnp.concatenate([sq, jnp.full((NP - N,), BIG, jnp.float32)])

    sig = jnp.linspace(0.5, 5.0, N_OUT).astype(jnp.float32)
    cof = -1.0 / (2.0 * sig * sig)

    cols, vals = _run(x0, x1, x2, sqp, cof)

    row = jnp.repeat(jnp.arange(N, dtype=jnp.int64), K)
    col = cols[: N * K].astype(jnp.int64)
    indices = jnp.stack([row, col], axis=0)
    values = vals[: N * K]
    return indices, values
